# baseline (speedup 1.0000x reference)
import jax
import jax.numpy as jnp
from jax import lax
from jax.experimental import pallas as pl
from jax.experimental.pallas import tpu as pltpu

_B, _P, _C = 32, 8732, 81
_RATIO = 3.0


def _dense_body(conf_ref, lab_ref, r_ref, qs_ref):
    x = conf_ref[0]
    e = jnp.exp(x - x[:, 0:1])
    ones = jnp.ones((1, _C), jnp.float32)
    r_ref[0] = lax.dot_general(ones, e, (((1,), (1,)), ((), ())),
                               preferred_element_type=jnp.float32)
    lab = lax.transpose(lab_ref[0], (1, 0))
    lab_i = lab.astype(jnp.int32)
    cls_iota = lax.broadcasted_iota(jnp.int32, (_P, _C), 1)
    m2 = (cls_iota == 0).astype(jnp.float32) - (cls_iota == lab_i).astype(
        jnp.float32)
    qs_ref[0, pl.program_id(0)] = jnp.sum(x * m2)


def _select_body(r_ref, q_ref, lab_ref, p4_ref, g4_ref, lab4_ref,
                 sl1_out, cls_out):
    r = r_ref[...]
    loss = jnp.log(r)
    lab = lab_ref[...]
    posm = lab > 0
    num_pos = jnp.sum(posm.astype(jnp.float32), axis=-1, keepdims=True)
    ce_pos = (jnp.sum(jnp.where(posm, loss, 0.0), axis=-1, keepdims=True)
              + q_ref[...])
    keys = jnp.where(posm, jnp.int32(-1),
                     lax.bitcast_convert_type(loss, jnp.int32))
    k_sel = jnp.where(num_pos > 0, _RATIO * num_pos, _RATIO)
    k_sel_i = k_sel.astype(jnp.int32)

    def bit_step(i, prefix):
        cand = prefix | lax.shift_left(jnp.int32(1), 30 - i)
        cnt = jnp.sum((keys >= cand).astype(jnp.int32), axis=-1,
                      keepdims=True)
        return jnp.where(cnt >= k_sel_i, cand, prefix)

    t = lax.fori_loop(0, 31, bit_step, jnp.zeros((_B, 1), jnp.int32))
    gt = keys > t
    cnt_gt = jnp.sum(gt.astype(jnp.float32), axis=-1, keepdims=True)
    sum_gt = jnp.sum(jnp.where(gt, loss, 0.0), axis=-1, keepdims=True)
    t_f = lax.bitcast_convert_type(t, jnp.float32)
    cls = ce_pos + sum_gt + (k_sel - cnt_gt) * t_f

    d = p4_ref[...] - g4_ref[...]
    ad = jnp.abs(d)
    sl1 = jnp.where(ad < 1.0, 0.5 * d * d, ad - 0.5)
    sl1_s = jnp.sum(jnp.where(lab4_ref[...] > 0, sl1, 0.0), axis=-1,
                    keepdims=True)

    total_pos = jnp.sum(num_pos)
    total_neg = _RATIO * jnp.sum((num_pos == 0).astype(jnp.float32))
    sl1_out[0] = jnp.sum(sl1_s) / jnp.maximum(total_pos, 1.0)
    cls_out[0] = jnp.sum(cls) / jnp.maximum(total_pos + total_neg, 1.0)


@jax.jit
def kernel(confidence, predicted_locations, labels, gt_locations):
    labels = labels.astype(jnp.int32)
    lab_f = labels.astype(jnp.float32).reshape(_B, 1, _P)
    r, q = pl.pallas_call(
        _dense_body,
        grid=(_B,),
        in_specs=[
            pl.BlockSpec((1, _P, _C), lambda b: (b, 0, 0)),
            pl.BlockSpec((1, 1, _P), lambda b: (b, 0, 0)),
        ],
        out_specs=[
            pl.BlockSpec((1, 1, _P), lambda b: (b, 0, 0)),
            pl.BlockSpec((1, _B), lambda b: (0, 0),
                         memory_space=pltpu.SMEM),
        ],
        out_shape=[
            jax.ShapeDtypeStruct((_B, 1, _P), jnp.float32),
            jax.ShapeDtypeStruct((1, _B), jnp.float32),
        ],
        compiler_params=pltpu.CompilerParams(
            dimension_semantics=("arbitrary",)),
    )(confidence, lab_f)

    lab4 = jnp.repeat(labels, 4, axis=1)
    p4 = predicted_locations.reshape(_B, 4 * _P)
    g4 = gt_locations.reshape(_B, 4 * _P)
    sl1_loss, cls_loss = pl.pallas_call(
        _select_body,
        in_specs=[
            pl.BlockSpec((_B, _P), lambda: (0, 0)),
            pl.BlockSpec((_B, 1), lambda: (0, 0)),
            pl.BlockSpec((_B, _P), lambda: (0, 0)),
            pl.BlockSpec((_B, 4 * _P), lambda: (0, 0)),
            pl.BlockSpec((_B, 4 * _P), lambda: (0, 0)),
            pl.BlockSpec((_B, 4 * _P), lambda: (0, 0)),
        ],
        out_specs=[
            pl.BlockSpec(memory_space=pltpu.SMEM),
            pl.BlockSpec(memory_space=pltpu.SMEM),
        ],
        out_shape=[
            jax.ShapeDtypeStruct((1,), jnp.float32),
            jax.ShapeDtypeStruct((1,), jnp.float32),
        ],
    )(r.reshape(_B, _P), q.reshape(_B, 1), labels, p4, g4, lab4)
    return (sl1_loss.reshape(()), cls_loss.reshape(()))


# speedup vs baseline: 1.0057x; 1.0057x over previous
import jax
import jax.numpy as jnp
from jax import lax
from jax.experimental import pallas as pl
from jax.experimental.pallas import tpu as pltpu

_B, _P, _C = 32, 8732, 81
_RATIO = 3.0


def _dense_body(conf_ref, lab_ref, r_ref, qs_ref):
    x = conf_ref[0]
    e = jnp.exp(x - x[:, 0:1])
    ones = jnp.ones((1, _C), jnp.float32)
    r_ref[0] = lax.dot_general(ones, e, (((1,), (1,)), ((), ())),
                               preferred_element_type=jnp.float32)
    lab = lax.transpose(lab_ref[0], (1, 0))
    lab_i = lab.astype(jnp.int32)
    cls_iota = lax.broadcasted_iota(jnp.int32, (_P, _C), 1)
    m2 = (cls_iota == 0).astype(jnp.float32) - (cls_iota == lab_i).astype(
        jnp.float32)
    li = lax.broadcasted_iota(jnp.int32, (1, 128), 1)
    qs_ref[0] = jnp.where(li == 0, jnp.sum(x * m2), 0.0)


def _select_body(r_ref, q_ref, lab_ref, p4_ref, g4_ref, lab4_ref,
                 sl1_out, cls_out):
    r = r_ref[...]
    loss = jnp.log(r)
    lab = lab_ref[...]
    posm = lab > 0
    num_pos = jnp.sum(posm.astype(jnp.float32), axis=-1, keepdims=True)
    ce_pos = (jnp.sum(jnp.where(posm, loss, 0.0), axis=-1, keepdims=True)
              + q_ref[:, 0:1])
    keys = jnp.where(posm, jnp.int32(-1),
                     lax.bitcast_convert_type(loss, jnp.int32))
    k_sel = jnp.where(num_pos > 0, _RATIO * num_pos, _RATIO)
    k_sel_i = k_sel.astype(jnp.int32)

    def bit_step(i, prefix):
        cand = prefix | lax.shift_left(jnp.int32(1), 30 - i)
        cnt = jnp.sum((keys >= cand).astype(jnp.int32), axis=-1,
                      keepdims=True)
        return jnp.where(cnt >= k_sel_i, cand, prefix)

    t = lax.fori_loop(0, 31, bit_step, jnp.zeros((_B, 1), jnp.int32))
    gt = keys > t
    cnt_gt = jnp.sum(gt.astype(jnp.float32), axis=-1, keepdims=True)
    sum_gt = jnp.sum(jnp.where(gt, loss, 0.0), axis=-1, keepdims=True)
    t_f = lax.bitcast_convert_type(t, jnp.float32)
    cls = ce_pos + sum_gt + (k_sel - cnt_gt) * t_f

    d = p4_ref[...] - g4_ref[...]
    ad = jnp.abs(d)
    sl1 = jnp.where(ad < 1.0, 0.5 * d * d, ad - 0.5)
    sl1_s = jnp.sum(jnp.where(lab4_ref[...] > 0, sl1, 0.0), axis=-1,
                    keepdims=True)

    total_pos = jnp.sum(num_pos)
    total_neg = _RATIO * jnp.sum((num_pos == 0).astype(jnp.float32))
    sl1_out[0] = jnp.sum(sl1_s) / jnp.maximum(total_pos, 1.0)
    cls_out[0] = jnp.sum(cls) / jnp.maximum(total_pos + total_neg, 1.0)


@jax.jit
def kernel(confidence, predicted_locations, labels, gt_locations):
    labels = labels.astype(jnp.int32)
    lab_f = labels.astype(jnp.float32).reshape(_B, 1, _P)
    r, q = pl.pallas_call(
        _dense_body,
        grid=(_B,),
        in_specs=[
            pl.BlockSpec((1, _P, _C), lambda b: (b, 0, 0)),
            pl.BlockSpec((1, 1, _P), lambda b: (b, 0, 0)),
        ],
        out_specs=[
            pl.BlockSpec((1, 1, _P), lambda b: (b, 0, 0)),
            pl.BlockSpec((1, 1, 128), lambda b: (b, 0, 0)),
        ],
        out_shape=[
            jax.ShapeDtypeStruct((_B, 1, _P), jnp.float32),
            jax.ShapeDtypeStruct((_B, 1, 128), jnp.float32),
        ],
        compiler_params=pltpu.CompilerParams(
            dimension_semantics=("parallel",)),
    )(confidence, lab_f)

    lab4 = jnp.repeat(labels, 4, axis=1)
    p4 = predicted_locations.reshape(_B, 4 * _P)
    g4 = gt_locations.reshape(_B, 4 * _P)
    sl1_loss, cls_loss = pl.pallas_call(
        _select_body,
        in_specs=[
            pl.BlockSpec((_B, _P), lambda: (0, 0)),
            pl.BlockSpec((_B, 128), lambda: (0, 0)),
            pl.BlockSpec((_B, _P), lambda: (0, 0)),
            pl.BlockSpec((_B, 4 * _P), lambda: (0, 0)),
            pl.BlockSpec((_B, 4 * _P), lambda: (0, 0)),
            pl.BlockSpec((_B, 4 * _P), lambda: (0, 0)),
        ],
        out_specs=[
            pl.BlockSpec(memory_space=pltpu.SMEM),
            pl.BlockSpec(memory_space=pltpu.SMEM),
        ],
        out_shape=[
            jax.ShapeDtypeStruct((1,), jnp.float32),
            jax.ShapeDtypeStruct((1,), jnp.float32),
        ],
    )(r.reshape(_B, _P), q.reshape(_B, 128), labels, p4, g4, lab4)
    return (sl1_loss.reshape(()), cls_loss.reshape(()))
